# Initial kernel scaffold; baseline (speedup 1.0000x reference)
import jax
import jax.numpy as jnp
from jax.experimental import pallas as pl


def kernel(x, edge_index, edge_attr, global_attr, params):
    raise NotImplementedError("write your pallas kernel here")


# speedup vs baseline: 3.2335x; 3.2335x over previous
import functools

import jax
import jax.numpy as jnp
from jax import lax
from jax.experimental import pallas as pl
from jax.experimental.pallas import tpu as pltpu
from jax.experimental.pallas import tpu_sc as plsc

F32 = jnp.float32
N = 10000
E = 160000
L = 32
EP4 = E // 4
NP4 = N // 4
BE = 2000
NPAD = 10240
NW = 32
CH = 128
CPW = 39


def _relu(x):
    return jnp.maximum(x, 0.0)


def _dot(a, b):
    return jnp.dot(a, b, preferred_element_type=F32)


def _dot_hi(a, b):
    return jnp.dot(a, b, preferred_element_type=F32,
                   precision=lax.Precision.HIGHEST)


def _ln_packed(h, mavg, scale, bias):
    mu = _dot_hi(h, mavg)
    hc = h - mu
    var = _dot_hi(hc * hc, mavg)
    return hc * lax.rsqrt(var + 1e-5) * scale + bias


def _fa(shape):
    n = len(shape)
    return pl.BlockSpec(shape, lambda i: (0,) * n)


def _sc_gather(row, col, pa, pb):
    mesh = plsc.VectorSubcoreMesh(core_axis_name="c", subcore_axis_name="s")

    @functools.partial(
        pl.kernel,
        mesh=mesh,
        out_type=(jax.ShapeDtypeStruct((E, L), F32),
                  jax.ShapeDtypeStruct((E, L), F32)),
        compiler_params=pltpu.CompilerParams(use_tc_tiling_on_sc=False),
        scratch_types=[
            pltpu.VMEM((CH,), jnp.int32),
            pltpu.VMEM((CH, L), F32),
            pltpu.SemaphoreType.DMA,
        ],
    )
    def k(row_h, col_h, pa_h, pb_h, sa_h, sb_h, ia, ba, sema):
        wid = lax.axis_index("s") * 2 + lax.axis_index("c")

        def chunk(base):
            pltpu.sync_copy(row_h.at[pl.ds(base, CH)], ia)
            pltpu.async_copy(pa_h.at[ia], ba, sema).wait()
            pltpu.sync_copy(ba, sa_h.at[pl.ds(base, CH)])
            pltpu.sync_copy(col_h.at[pl.ds(base, CH)], ia)
            pltpu.async_copy(pb_h.at[ia], ba, sema).wait()
            pltpu.sync_copy(ba, sb_h.at[pl.ds(base, CH)])

        def body(j, carry):
            chunk(wid * (CPW * CH) + j * CH)
            return carry

        lax.fori_loop(0, CPW, body, 0)

        @pl.when(wid < 2)
        def _():
            chunk((NW * CPW + wid) * CH)

    return k(row, col, pa, pb)


def _sc_scatter(vals, col, zeros):
    mesh = plsc.VectorSubcoreMesh(core_axis_name="c", subcore_axis_name="s")

    @functools.partial(
        pl.kernel,
        mesh=mesh,
        out_type=jax.ShapeDtypeStruct((2, NPAD, L), F32),
        compiler_params=pltpu.CompilerParams(use_tc_tiling_on_sc=False),
        scratch_types=[
            pltpu.VMEM((CH,), jnp.int32),
            pltpu.VMEM((CH, L), F32),
            pltpu.VMEM_SHARED((NPAD, L), F32),
        ],
    )
    def k(vals_h, col_h, z_h, out_h, idx, buf, acc):
        c = lax.axis_index("c")
        s = lax.axis_index("s")
        pltpu.sync_copy(z_h, acc.at[pl.ds(s * 640, 640)])
        plsc.subcore_barrier()

        def chunk(base):
            pltpu.sync_copy(col_h.at[pl.ds(base, CH)], idx)
            pltpu.sync_copy(vals_h.at[pl.ds(base, CH)], buf)
            pltpu.sync_copy(buf, acc.at[idx], add=True)

        def body(j, carry):
            chunk(c * (E // 2) + s * (CPW * CH) + j * CH)
            return carry

        lax.fori_loop(0, CPW, body, 0)
        @pl.when(s == 15)
        def _():
            chunk(c * (E // 2) + 16 * CPW * CH)
        plsc.subcore_barrier()
        pltpu.sync_copy(acc.at[pl.ds(s * 640, 640)], out_h.at[c, pl.ds(s * 640, 640)])

    return k(vals, col, zeros)


def _enc_node_call(x, cnt2, ga, w, interpret=False):
    def body(x_r, cnt_r, ga_r,
             enW1, enb1, enW2, enb2, enls, enlb,
             egW1, egb1, egW2, egb2,
             A1, A2, B1, B2, N1, D1, D2, b1e, Tt,
             v0_r, pa0_r, pb0_r, nv0_r, PA1_r, PB1_r, inv_r, g0_r, cb1_r):
        h = _relu(_dot(x_r[...], enW1[...]) + enb1[...])
        h = _dot(h, enW2[...]) + enb2[...]
        h = _relu(h)
        mu = jnp.mean(h, axis=1, keepdims=True)
        hc = h - mu
        var = jnp.mean(hc * hc, axis=1, keepdims=True)
        v0 = hc * lax.rsqrt(var + 1e-5) * enls[...] + enlb[...]
        v0_r[...] = v0
        g = _relu(_dot(ga_r[...], egW1[...]) + egb1[...])
        g0 = _relu(_dot(g, egW2[...]) + egb2[...])
        g0_r[...] = g0
        pa0 = _dot(v0, A1[...])
        pb0 = _dot(v0, B1[...])
        pa0_r[...] = pa0
        pb0_r[...] = pb0
        nv0_r[...] = _dot(v0, N1[...])
        PA1_r[...] = pa0 + _dot(v0, A2[...])
        PB1_r[...] = pb0 + _dot(v0, B2[...])
        cnt = cnt_r[0, :, 0:1] + cnt_r[1, :, 0:1]
        inv_r[...] = jnp.broadcast_to(1.0 / jnp.maximum(cnt, 1.0), inv_r.shape)
        cb1_r[...] = _dot_hi(
            _dot(g0, D1[...]) + _dot(g0, D2[...]) + b1e[...], Tt[...])

    BN = 2000
    grid = (N // BN,)
    in_specs = ([pl.BlockSpec((BN, 128), lambda i: (i, 0)),
                 pl.BlockSpec((2, BN, L), lambda i: (0, i, 0)),
                 _fa(ga.shape)] + [_fa(a.shape) for a in w])
    out_specs = [pl.BlockSpec((BN, L), lambda i: (i, 0))] * 7 + [
        pl.BlockSpec((1, L), lambda i: (0, 0)),
        pl.BlockSpec((1, 128), lambda i: (0, 0)),
    ]
    outs = [jax.ShapeDtypeStruct((N, L), F32)] * 7 + [
        jax.ShapeDtypeStruct((1, L), F32),
        jax.ShapeDtypeStruct((1, 128), F32),
    ]
    return pl.pallas_call(body, grid=grid, in_specs=in_specs,
                          out_specs=out_specs, out_shape=outs,
                          interpret=interpret)(x, cnt2, ga, *w)


def _enc_edge_call(eap, w, interpret=False):
    def body(ea_r, W1p, b1t, W2p, b2t, lst, lbt, Mavg, C1p, e0_r, ec0_r):
        h = _relu(_dot(ea_r[...], W1p[...]) + b1t[...])
        h = _dot(h, W2p[...]) + b2t[...]
        h = _relu(h)
        e0 = _ln_packed(h, Mavg[...], lst[...], lbt[...])
        e0_r[...] = e0
        ec0_r[...] = _dot(e0, C1p[...])

    grid = (EP4 // BE,)
    in_specs = [pl.BlockSpec((BE, 64), lambda i: (i, 0))] + [_fa(a.shape) for a in w]
    out_specs = [pl.BlockSpec((BE, 128), lambda i: (i, 0))] * 2
    outs = [jax.ShapeDtypeStruct((EP4, 128), F32)] * 2
    return pl.pallas_call(body, grid=grid, in_specs=in_specs,
                          out_specs=out_specs, out_shape=outs,
                          interpret=interpret)(eap, *w)


def _edge_step_call(ep, ec0p, sap, sbp, cb, w, interpret=False):
    def body(ep_r, ec0_r, sa_r, sb_r, cb_r,
             C2p, W2p, b2t, lst, lbt, dW1p, db1t, dW2p, db2t, dlst, dlbt, Mavg,
             ecp_r, enp_r, esum_r):
        t = _dot(ep_r[...], C2p[...]) + ec0_r[...] + sa_r[...] + sb_r[...] + cb_r[...]
        h1 = _relu(t)
        h2 = _dot(h1, W2p[...]) + b2t[...]
        h2 = _relu(h2)
        ec = _ln_packed(h2, Mavg[...], lst[...], lbt[...])
        ecp_r[...] = ec

        @pl.when(pl.program_id(0) == 0)
        def _():
            esum_r[...] = jnp.zeros_like(esum_r)

        esum_r[...] += jnp.sum(ec, axis=0, keepdims=True)
        d = _relu(_dot(ec, dW1p[...]) + db1t[...])
        d = _dot(d, dW2p[...]) + db2t[...]
        d = _relu(d)
        enp_r[...] = _ln_packed(d, Mavg[...], dlst[...], dlbt[...])

    grid = (EP4 // BE,)
    in_specs = ([pl.BlockSpec((BE, 128), lambda i: (i, 0))] * 4
                + [_fa(cb.shape)] + [_fa(a.shape) for a in w])
    out_specs = [pl.BlockSpec((BE, 128), lambda i: (i, 0))] * 2 + [
        pl.BlockSpec((1, 128), lambda i: (0, 0))]
    outs = [jax.ShapeDtypeStruct((EP4, 128), F32)] * 2 + [
        jax.ShapeDtypeStruct((1, 128), F32)]
    return pl.pallas_call(body, grid=grid, in_specs=in_specs,
                          out_specs=out_specs, out_shape=outs,
                          interpret=interpret)(ep, ec0p, sap, sbp, cb, *w)


def _node_step_call(vp, nv0p, pa0p, pb0p, agg2, invp, esum, g0, g, w,
                    interpret=False):
    def body(vp_r, nv0_r, pa0_r, pb0_r, agg2_r, inv_r, esum_r, g0_r, g_r,
             N2p, N3p, nW2p, nb2t, nlst, nlbt,
             dnW1p, dnb1t, dnW2p, dnb2t, dnlst, dnlbt,
             Mavg, Pfold, Tt,
             Ng1, Ng2, b1n, G1, G2, G3, G4, b1g, gW2, gb2,
             dgW1, dgb1, dgW2, dgb2, D1, D2, b1e, A2p, B2p,
             vn_r, PA_r, PB_r, gn_r, cbn_r):
        g0 = g0_r[...]
        g = g_r[...]
        agg = (agg2_r[0] + agg2_r[1]) * inv_r[...]
        gb = _dot_hi(_dot(g0, Ng1[...]) + _dot(g, Ng2[...]) + b1n[...], Tt[...])
        n1 = nv0_r[...] + _dot(vp_r[...], N2p[...]) + _dot(agg, N3p[...]) + gb
        h = _relu(n1)
        h2 = _dot(h, nW2p[...]) + nb2t[...]
        h2 = _relu(h2)
        vc = _ln_packed(h2, Mavg[...], nlst[...], nlbt[...])
        vsum = jnp.sum(vc, axis=0, keepdims=True)
        meanv = _dot_hi(vsum, Pfold[...]) / N
        meane = _dot_hi(esum_r[...], Pfold[...]) / E
        g1 = (_dot(g0, G1[...]) + _dot(g, G2[...]) + _dot(meanv, G3[...])
              + _dot(meane, G4[...]) + b1g[...])
        gc = _relu(_dot(_relu(g1), gW2[...]) + gb2[...])
        gn = _relu(_dot(_relu(_dot(gc, dgW1[...]) + dgb1[...]), dgW2[...])
                   + dgb2[...])
        gn_r[...] = gn
        d = _relu(_dot(vc, dnW1p[...]) + dnb1t[...])
        d = _dot(d, dnW2p[...]) + dnb2t[...]
        d = _relu(d)
        vn = _ln_packed(d, Mavg[...], dnlst[...], dnlbt[...])
        vn_r[...] = vn
        PA_r[...] = pa0_r[...] + _dot(vn, A2p[...])
        PB_r[...] = pb0_r[...] + _dot(vn, B2p[...])
        cbn_r[...] = _dot_hi(_dot(g0, D1[...]) + _dot(gn, D2[...]) + b1e[...],
                             Tt[...])

    outs = [jax.ShapeDtypeStruct((NP4, 128), F32)] * 3 + [
        jax.ShapeDtypeStruct((1, L), F32),
        jax.ShapeDtypeStruct((1, 128), F32),
    ]
    return pl.pallas_call(body, out_shape=outs, interpret=interpret)(
        vp, nv0p, pa0p, pb0p, agg2, invp, esum, g0, g, *w)


def _out_node_call(v, g, w, interpret=False):
    def body(v_r, g_r, W1, b1, W2, b2, gW1, gb1, gW2, gb2, ov_r, og_r):
        h = _relu(_dot(v_r[...], W1[...]) + b1[...])
        ov_r[...] = _dot(h, W2[...]) + b2[...]
        hg = _relu(_dot(g_r[...], gW1[...]) + gb1[...])
        og_r[...] = _dot(hg, gW2[...]) + gb2[...]

    outs = [jax.ShapeDtypeStruct((N, 128), F32),
            jax.ShapeDtypeStruct((1, 16), F32)]
    return pl.pallas_call(body, out_shape=outs, interpret=interpret)(v, g, *w)


def _out_edge_call(ep, w, interpret=False):
    def body(ep_r, W1p, b1t, W2p, b2t, oe_r):
        h = _relu(_dot(ep_r[...], W1p[...]) + b1t[...])
        oe_r[...] = _dot(h, W2p[...]) + b2t[...]

    grid = (EP4 // BE,)
    in_specs = [pl.BlockSpec((BE, 128), lambda i: (i, 0))] + [
        _fa(a.shape) for a in w]
    out_specs = [pl.BlockSpec((BE, 64), lambda i: (i, 0))]
    outs = [jax.ShapeDtypeStruct((EP4, 64), F32)]
    return pl.pallas_call(body, grid=grid, in_specs=in_specs,
                          out_specs=out_specs, out_shape=outs,
                          interpret=interpret)(ep, *w)


def _bd4(wmat):
    return jax.scipy.linalg.block_diag(wmat, wmat, wmat, wmat)


def _t4(b):
    return jnp.tile(jnp.reshape(b, (1, -1)), (1, 4))


def _forward(x, edge_index, edge_attr, global_attr, params, interpret=False):
    p = params
    row = edge_index[0]
    col = edge_index[1]
    eye = jnp.eye(L, dtype=F32)
    Tt = jnp.concatenate([eye] * 4, axis=1)
    Pfold = jnp.concatenate([eye] * 4, axis=0)
    Mavg = _bd4(jnp.full((L, L), 1.0 / L, F32))

    We = p["core_edge"]["W1"]
    A1, A2 = We[0:32], We[32:64]
    B1, B2 = We[64:96], We[96:128]
    C1, C2 = We[128:160], We[160:192]
    D1, D2 = We[192:224], We[224:256]
    b1e = jnp.reshape(p["core_edge"]["b1"], (1, L))
    Wn = p["core_node"]["W1"]
    N1, N2, N3, Ng1, Ng2 = Wn[0:32], Wn[32:64], Wn[64:96], Wn[96:128], Wn[128:160]
    b1n = jnp.reshape(p["core_node"]["b1"], (1, L))
    Wg = p["core_glob"]["W1"]
    G1, G2, G3, G4 = Wg[0:32], Wg[32:64], Wg[64:96], Wg[96:128]

    def r1(a):
        return jnp.reshape(a, (1, -1))

    zeros640 = jnp.zeros((640, L), F32)
    ones_e = jnp.ones((E, L), F32)
    cntp = _sc_scatter(ones_e, col, zeros640)
    cnt2 = cntp[:, :N, :]

    enc_w = [
        p["enc_node"]["W1"], r1(p["enc_node"]["b1"]),
        p["enc_node"]["W2"], r1(p["enc_node"]["b2"]),
        r1(p["enc_node"]["ln_scale"]), r1(p["enc_node"]["ln_bias"]),
        p["enc_glob"]["W1"], r1(p["enc_glob"]["b1"]),
        p["enc_glob"]["W2"], r1(p["enc_glob"]["b2"]),
        A1, A2, B1, B2, N1, D1, D2, b1e, Tt,
    ]
    (v0, pa0, pb0, nv0, PA, PB, invb, g0, cb) = _enc_node_call(
        x, cnt2, global_attr, enc_w, interpret=interpret)

    ee_w = [
        _bd4(p["enc_edge"]["W1"]), _t4(p["enc_edge"]["b1"]),
        _bd4(p["enc_edge"]["W2"]), _t4(p["enc_edge"]["b2"]),
        _t4(p["enc_edge"]["ln_scale"]), _t4(p["enc_edge"]["ln_bias"]),
        Mavg, _bd4(C1),
    ]
    eap = jnp.reshape(edge_attr, (EP4, 64))
    e0p, ec0p = _enc_edge_call(eap, ee_w, interpret=interpret)

    es_w = [
        _bd4(C2), _bd4(p["core_edge"]["W2"]), _t4(p["core_edge"]["b2"]),
        _t4(p["core_edge"]["ln_scale"]), _t4(p["core_edge"]["ln_bias"]),
        _bd4(p["dec_edge"]["W1"]), _t4(p["dec_edge"]["b1"]),
        _bd4(p["dec_edge"]["W2"]), _t4(p["dec_edge"]["b2"]),
        _t4(p["dec_edge"]["ln_scale"]), _t4(p["dec_edge"]["ln_bias"]),
        Mavg,
    ]
    ns_w = [
        _bd4(N2), _bd4(N3), _bd4(p["core_node"]["W2"]), _t4(p["core_node"]["b2"]),
        _t4(p["core_node"]["ln_scale"]), _t4(p["core_node"]["ln_bias"]),
        _bd4(p["dec_node"]["W1"]), _t4(p["dec_node"]["b1"]),
        _bd4(p["dec_node"]["W2"]), _t4(p["dec_node"]["b2"]),
        _t4(p["dec_node"]["ln_scale"]), _t4(p["dec_node"]["ln_bias"]),
        Mavg, Pfold, Tt,
        Ng1, Ng2, b1n, G1, G2, G3, G4, r1(p["core_glob"]["b1"]),
        p["core_glob"]["W2"], r1(p["core_glob"]["b2"]),
        p["dec_glob"]["W1"], r1(p["dec_glob"]["b1"]),
        p["dec_glob"]["W2"], r1(p["dec_glob"]["b2"]),
        D1, D2, b1e, _bd4(A2), _bd4(B2),
    ]

    nv0p = jnp.reshape(nv0, (NP4, 128))
    pa0p = jnp.reshape(pa0, (NP4, 128))
    pb0p = jnp.reshape(pb0, (NP4, 128))
    invp = jnp.reshape(invb, (NP4, 128))
    vp = jnp.reshape(v0, (NP4, 128))
    ep = e0p
    g = g0
    for _ in range(3):
        sa, sb = _sc_gather(row, col, jnp.reshape(PA, (N, L)),
                            jnp.reshape(PB, (N, L)))
        sap = jnp.reshape(sa, (EP4, 128))
        sbp = jnp.reshape(sb, (EP4, 128))
        ecp, enp, esum = _edge_step_call(ep, ec0p, sap, sbp, cb, es_w,
                                         interpret=interpret)
        aggp = _sc_scatter(jnp.reshape(ecp, (E, L)), col, zeros640)
        agg2 = jnp.reshape(aggp[:, :N, :], (2, NP4, 128))
        vp, PA, PB, g, cb = _node_step_call(
            vp, nv0p, pa0p, pb0p, agg2, invp, esum, g0, g, ns_w,
            interpret=interpret)
        ep = enp

    on_w = [
        p["out_node"]["W1"], r1(p["out_node"]["b1"]),
        p["out_node"]["W2"], r1(p["out_node"]["b2"]),
        p["out_glob"]["W1"], r1(p["out_glob"]["b1"]),
        p["out_glob"]["W2"], r1(p["out_glob"]["b2"]),
    ]
    out_v, out_g = _out_node_call(jnp.reshape(vp, (N, L)), g, on_w,
                                  interpret=interpret)
    oe_w = [
        _bd4(p["out_edge"]["W1"]), _t4(p["out_edge"]["b1"]),
        _bd4(p["out_edge"]["W2"]), _t4(p["out_edge"]["b2"]),
    ]
    (oep,) = _out_edge_call(ep, oe_w, interpret=interpret)
    out_e = jnp.reshape(oep, (E, 16))
    return (out_v, out_e, out_g)


def kernel(x, edge_index, edge_attr, global_attr, params):
    return _forward(x, edge_index, edge_attr, global_attr, params)


# speedup vs baseline: 3.9968x; 1.2360x over previous
import functools

import jax
import jax.numpy as jnp
from jax import lax
from jax.experimental import pallas as pl
from jax.experimental.pallas import tpu as pltpu
from jax.experimental.pallas import tpu_sc as plsc

F32 = jnp.float32
N = 10000
E = 160000
L = 32
EP4 = E // 4
NP4 = N // 4
BE = 2000
NPAD = 10240
NW = 32
CH = 128
CPW = 39


def _relu(x):
    return jnp.maximum(x, 0.0)


def _dot(a, b):
    return jnp.dot(a, b, preferred_element_type=F32)


def _dot_hi(a, b):
    return jnp.dot(a, b, preferred_element_type=F32,
                   precision=lax.Precision.HIGHEST)


def _ln_packed(h, mavg, scale, bias):
    mu = _dot_hi(h, mavg)
    hc = h - mu
    var = _dot_hi(hc * hc, mavg)
    return hc * lax.rsqrt(var + 1e-5) * scale + bias


def _fa(shape):
    n = len(shape)
    return pl.BlockSpec(shape, lambda i: (0,) * n)


GG = 10


def _sc_gather(row2d, col2d, pa, pb):
    mesh = plsc.VectorSubcoreMesh(core_axis_name="c", subcore_axis_name="s")

    @functools.partial(
        pl.kernel,
        mesh=mesh,
        out_type=(jax.ShapeDtypeStruct((E, L), F32),
                  jax.ShapeDtypeStruct((E, L), F32)),
        compiler_params=pltpu.CompilerParams(use_tc_tiling_on_sc=False),
        scratch_types=[
            pltpu.VMEM((CPW + 1, CH), jnp.int32),
            pltpu.VMEM((GG * CH, L), F32),
            pltpu.VMEM((GG * CH, L), F32),
            pltpu.SemaphoreType.DMA,
            pltpu.SemaphoreType.DMA,
        ],
    )
    def k(row_h, col_h, pa_h, pb_h, sa_h, sb_h, idx, b0, b1, gsem, wsem):
        wid = lax.axis_index("s") * 2 + lax.axis_index("c")
        bufs = (b0, b1)

        def phase(src_h, tab_h, out_h, pending):
            pltpu.sync_copy(src_h.at[pl.ds(wid * CPW, CPW)],
                            idx.at[pl.ds(0, CPW)])

            @pl.when(wid < 2)
            def _():
                pltpu.sync_copy(src_h.at[pl.ds(NW * CPW + wid, 1)],
                                idx.at[pl.ds(CPW, 1)])
            for g in range((CPW + GG - 1) // GG):
                nch = min(GG, CPW - g * GG)
                buf = bufs[g % 2]
                copies = []
                for j in range(nch):
                    ch = g * GG + j
                    copies.append(pltpu.async_copy(
                        tab_h.at[idx.at[ch]],
                        buf.at[pl.ds(j * CH, CH)], gsem))
                if pending[0] is not None:
                    pending[0].wait()
                    pending[0] = None
                for cpy in copies:
                    cpy.wait()
                wb = pltpu.async_copy(
                    buf.at[pl.ds(0, nch * CH)],
                    out_h.at[pl.ds(wid * (CPW * CH) + g * GG * CH, nch * CH)],
                    wsem)
                pending[0] = wb

            @pl.when(wid < 2)
            def _():
                cpy = pltpu.async_copy(tab_h.at[idx.at[CPW]],
                                       bufs[0].at[pl.ds(0, CH)], gsem)
                cpy.wait()
                pltpu.sync_copy(bufs[0].at[pl.ds(0, CH)],
                                out_h.at[pl.ds((NW * CPW + wid) * CH, CH)])
            if pending[0] is not None:
                pending[0].wait()
                pending[0] = None

        pending = [None]
        phase(row_h, pa_h, sa_h, pending)
        phase(col_h, pb_h, sb_h, pending)

    return k(row2d, col2d, pa, pb)


SG = 20


def _sc_scatter(vals, col2d, zeros):
    mesh = plsc.VectorSubcoreMesh(core_axis_name="c", subcore_axis_name="s")

    @functools.partial(
        pl.kernel,
        mesh=mesh,
        out_type=jax.ShapeDtypeStruct((2, NPAD, L), F32),
        compiler_params=pltpu.CompilerParams(use_tc_tiling_on_sc=False),
        scratch_types=[
            pltpu.VMEM((CPW + 1, CH), jnp.int32),
            pltpu.VMEM((SG * CH, L), F32),
            pltpu.VMEM_SHARED((NPAD, L), F32),
            pltpu.SemaphoreType.DMA,
        ],
    )
    def k(vals_h, col_h, z_h, out_h, idx, buf, acc, ssem):
        c = lax.axis_index("c")
        s = lax.axis_index("s")
        base_ch = c * 625 + s * CPW
        pltpu.sync_copy(z_h, acc.at[pl.ds(s * 640, 640)])
        pltpu.sync_copy(col_h.at[pl.ds(base_ch, CPW)], idx.at[pl.ds(0, CPW)])

        @pl.when(s == 15)
        def _():
            pltpu.sync_copy(col_h.at[pl.ds(c * 625 + 624, 1)],
                            idx.at[pl.ds(CPW, 1)])
        plsc.subcore_barrier()
        for g in range((CPW + SG - 1) // SG):
            nch = min(SG, CPW - g * SG)
            pltpu.sync_copy(
                vals_h.at[pl.ds((base_ch + g * SG) * CH, nch * CH)],
                buf.at[pl.ds(0, nch * CH)])
            copies = []
            for j in range(nch):
                copies.append(pltpu.async_copy(
                    buf.at[pl.ds(j * CH, CH)],
                    acc.at[idx.at[g * SG + j]], ssem, add=True))
            for cpy in copies:
                cpy.wait()

        @pl.when(s == 15)
        def _():
            pltpu.sync_copy(vals_h.at[pl.ds((c * 625 + 624) * CH, CH)],
                            buf.at[pl.ds(0, CH)])
            pltpu.sync_copy(buf.at[pl.ds(0, CH)], acc.at[idx.at[CPW]], add=True)
        plsc.subcore_barrier()
        pltpu.sync_copy(acc.at[pl.ds(s * 640, 640)], out_h.at[c, pl.ds(s * 640, 640)])

    return k(vals, col2d, zeros)


def _enc_node_call(x, cnt2, ga, w, interpret=False):
    def body(x_r, cnt_r, ga_r,
             enW1, enb1, enW2, enb2, enls, enlb,
             egW1, egb1, egW2, egb2,
             A1, A2, B1, B2, N1, D1, D2, b1e, Tt,
             v0_r, pa0_r, pb0_r, nv0_r, PA1_r, PB1_r, inv_r, g0_r, cb1_r):
        h = _relu(_dot(x_r[...], enW1[...]) + enb1[...])
        h = _dot(h, enW2[...]) + enb2[...]
        h = _relu(h)
        mu = jnp.mean(h, axis=1, keepdims=True)
        hc = h - mu
        var = jnp.mean(hc * hc, axis=1, keepdims=True)
        v0 = hc * lax.rsqrt(var + 1e-5) * enls[...] + enlb[...]
        v0_r[...] = v0
        g = _relu(_dot(ga_r[...], egW1[...]) + egb1[...])
        g0 = _relu(_dot(g, egW2[...]) + egb2[...])
        g0_r[...] = g0
        pa0 = _dot(v0, A1[...])
        pb0 = _dot(v0, B1[...])
        pa0_r[...] = pa0
        pb0_r[...] = pb0
        nv0_r[...] = _dot(v0, N1[...])
        PA1_r[...] = pa0 + _dot(v0, A2[...])
        PB1_r[...] = pb0 + _dot(v0, B2[...])
        cnt = cnt_r[0, :, 0:1] + cnt_r[1, :, 0:1]
        inv_r[...] = jnp.broadcast_to(1.0 / jnp.maximum(cnt, 1.0), inv_r.shape)
        cb1_r[...] = _dot_hi(
            _dot(g0, D1[...]) + _dot(g0, D2[...]) + b1e[...], Tt[...])

    BN = 2000
    grid = (N // BN,)
    in_specs = ([pl.BlockSpec((BN, 128), lambda i: (i, 0)),
                 pl.BlockSpec((2, BN, L), lambda i: (0, i, 0)),
                 _fa(ga.shape)] + [_fa(a.shape) for a in w])
    out_specs = [pl.BlockSpec((BN, L), lambda i: (i, 0))] * 7 + [
        pl.BlockSpec((1, L), lambda i: (0, 0)),
        pl.BlockSpec((1, 128), lambda i: (0, 0)),
    ]
    outs = [jax.ShapeDtypeStruct((N, L), F32)] * 7 + [
        jax.ShapeDtypeStruct((1, L), F32),
        jax.ShapeDtypeStruct((1, 128), F32),
    ]
    return pl.pallas_call(body, grid=grid, in_specs=in_specs,
                          out_specs=out_specs, out_shape=outs,
                          interpret=interpret)(x, cnt2, ga, *w)


def _enc_edge_call(eap, w, interpret=False):
    def body(ea_r, W1p, b1t, W2p, b2t, lst, lbt, Mavg, C1p, e0_r, ec0_r):
        h = _relu(_dot(ea_r[...], W1p[...]) + b1t[...])
        h = _dot(h, W2p[...]) + b2t[...]
        h = _relu(h)
        e0 = _ln_packed(h, Mavg[...], lst[...], lbt[...])
        e0_r[...] = e0
        ec0_r[...] = _dot(e0, C1p[...])

    grid = (EP4 // BE,)
    in_specs = [pl.BlockSpec((BE, 64), lambda i: (i, 0))] + [_fa(a.shape) for a in w]
    out_specs = [pl.BlockSpec((BE, 128), lambda i: (i, 0))] * 2
    outs = [jax.ShapeDtypeStruct((EP4, 128), F32)] * 2
    return pl.pallas_call(body, grid=grid, in_specs=in_specs,
                          out_specs=out_specs, out_shape=outs,
                          interpret=interpret)(eap, *w)


def _edge_step_call(ep, ec0p, sap, sbp, cb, w, interpret=False):
    def body(ep_r, ec0_r, sa_r, sb_r, cb_r,
             C2p, W2p, b2t, lst, lbt, dW1p, db1t, dW2p, db2t, dlst, dlbt, Mavg,
             ecp_r, enp_r, esum_r):
        t = _dot(ep_r[...], C2p[...]) + ec0_r[...] + sa_r[...] + sb_r[...] + cb_r[...]
        h1 = _relu(t)
        h2 = _dot(h1, W2p[...]) + b2t[...]
        h2 = _relu(h2)
        ec = _ln_packed(h2, Mavg[...], lst[...], lbt[...])
        ecp_r[...] = ec

        @pl.when(pl.program_id(0) == 0)
        def _():
            esum_r[...] = jnp.zeros_like(esum_r)

        esum_r[...] += jnp.sum(ec, axis=0, keepdims=True)
        d = _relu(_dot(ec, dW1p[...]) + db1t[...])
        d = _dot(d, dW2p[...]) + db2t[...]
        d = _relu(d)
        enp_r[...] = _ln_packed(d, Mavg[...], dlst[...], dlbt[...])

    grid = (EP4 // BE,)
    in_specs = ([pl.BlockSpec((BE, 128), lambda i: (i, 0))] * 4
                + [_fa(cb.shape)] + [_fa(a.shape) for a in w])
    out_specs = [pl.BlockSpec((BE, 128), lambda i: (i, 0))] * 2 + [
        pl.BlockSpec((1, 128), lambda i: (0, 0))]
    outs = [jax.ShapeDtypeStruct((EP4, 128), F32)] * 2 + [
        jax.ShapeDtypeStruct((1, 128), F32)]
    return pl.pallas_call(body, grid=grid, in_specs=in_specs,
                          out_specs=out_specs, out_shape=outs,
                          interpret=interpret)(ep, ec0p, sap, sbp, cb, *w)


def _node_step_call(vp, nv0p, pa0p, pb0p, agg2, invp, esum, g0, g, w,
                    interpret=False):
    def body(vp_r, nv0_r, pa0_r, pb0_r, agg2_r, inv_r, esum_r, g0_r, g_r,
             N2p, N3p, nW2p, nb2t, nlst, nlbt,
             dnW1p, dnb1t, dnW2p, dnb2t, dnlst, dnlbt,
             Mavg, Pfold, Tt,
             Ng1, Ng2, b1n, G1, G2, G3, G4, b1g, gW2, gb2,
             dgW1, dgb1, dgW2, dgb2, D1, D2, b1e, A2p, B2p,
             vn_r, PA_r, PB_r, gn_r, cbn_r):
        g0 = g0_r[...]
        g = g_r[...]
        agg = (agg2_r[0] + agg2_r[1]) * inv_r[...]
        gb = _dot_hi(_dot(g0, Ng1[...]) + _dot(g, Ng2[...]) + b1n[...], Tt[...])
        n1 = nv0_r[...] + _dot(vp_r[...], N2p[...]) + _dot(agg, N3p[...]) + gb
        h = _relu(n1)
        h2 = _dot(h, nW2p[...]) + nb2t[...]
        h2 = _relu(h2)
        vc = _ln_packed(h2, Mavg[...], nlst[...], nlbt[...])
        vsum = jnp.sum(vc, axis=0, keepdims=True)
        meanv = _dot_hi(vsum, Pfold[...]) / N
        meane = _dot_hi(esum_r[...], Pfold[...]) / E
        g1 = (_dot(g0, G1[...]) + _dot(g, G2[...]) + _dot(meanv, G3[...])
              + _dot(meane, G4[...]) + b1g[...])
        gc = _relu(_dot(_relu(g1), gW2[...]) + gb2[...])
        gn = _relu(_dot(_relu(_dot(gc, dgW1[...]) + dgb1[...]), dgW2[...])
                   + dgb2[...])
        gn_r[...] = gn
        d = _relu(_dot(vc, dnW1p[...]) + dnb1t[...])
        d = _dot(d, dnW2p[...]) + dnb2t[...]
        d = _relu(d)
        vn = _ln_packed(d, Mavg[...], dnlst[...], dnlbt[...])
        vn_r[...] = vn
        PA_r[...] = pa0_r[...] + _dot(vn, A2p[...])
        PB_r[...] = pb0_r[...] + _dot(vn, B2p[...])
        cbn_r[...] = _dot_hi(_dot(g0, D1[...]) + _dot(gn, D2[...]) + b1e[...],
                             Tt[...])

    outs = [jax.ShapeDtypeStruct((NP4, 128), F32)] * 3 + [
        jax.ShapeDtypeStruct((1, L), F32),
        jax.ShapeDtypeStruct((1, 128), F32),
    ]
    return pl.pallas_call(body, out_shape=outs, interpret=interpret)(
        vp, nv0p, pa0p, pb0p, agg2, invp, esum, g0, g, *w)


def _out_node_call(v, g, w, interpret=False):
    def body(v_r, g_r, W1, b1, W2, b2, gW1, gb1, gW2, gb2, ov_r, og_r):
        h = _relu(_dot(v_r[...], W1[...]) + b1[...])
        ov_r[...] = _dot(h, W2[...]) + b2[...]
        hg = _relu(_dot(g_r[...], gW1[...]) + gb1[...])
        og_r[...] = _dot(hg, gW2[...]) + gb2[...]

    outs = [jax.ShapeDtypeStruct((N, 128), F32),
            jax.ShapeDtypeStruct((1, 16), F32)]
    return pl.pallas_call(body, out_shape=outs, interpret=interpret)(v, g, *w)


def _out_edge_call(ep, w, interpret=False):
    def body(ep_r, W1p, b1t, W2p, b2t, oe_r):
        h = _relu(_dot(ep_r[...], W1p[...]) + b1t[...])
        oe_r[...] = _dot(h, W2p[...]) + b2t[...]

    grid = (EP4 // BE,)
    in_specs = [pl.BlockSpec((BE, 128), lambda i: (i, 0))] + [
        _fa(a.shape) for a in w]
    out_specs = [pl.BlockSpec((BE, 64), lambda i: (i, 0))]
    outs = [jax.ShapeDtypeStruct((EP4, 64), F32)]
    return pl.pallas_call(body, grid=grid, in_specs=in_specs,
                          out_specs=out_specs, out_shape=outs,
                          interpret=interpret)(ep, *w)


def _bd4(wmat):
    return jax.scipy.linalg.block_diag(wmat, wmat, wmat, wmat)


def _t4(b):
    return jnp.tile(jnp.reshape(b, (1, -1)), (1, 4))


def _forward(x, edge_index, edge_attr, global_attr, params, interpret=False):
    p = params
    row = edge_index[0]
    col = edge_index[1]
    eye = jnp.eye(L, dtype=F32)
    Tt = jnp.concatenate([eye] * 4, axis=1)
    Pfold = jnp.concatenate([eye] * 4, axis=0)
    Mavg = _bd4(jnp.full((L, L), 1.0 / L, F32))

    We = p["core_edge"]["W1"]
    A1, A2 = We[0:32], We[32:64]
    B1, B2 = We[64:96], We[96:128]
    C1, C2 = We[128:160], We[160:192]
    D1, D2 = We[192:224], We[224:256]
    b1e = jnp.reshape(p["core_edge"]["b1"], (1, L))
    Wn = p["core_node"]["W1"]
    N1, N2, N3, Ng1, Ng2 = Wn[0:32], Wn[32:64], Wn[64:96], Wn[96:128], Wn[128:160]
    b1n = jnp.reshape(p["core_node"]["b1"], (1, L))
    Wg = p["core_glob"]["W1"]
    G1, G2, G3, G4 = Wg[0:32], Wg[32:64], Wg[64:96], Wg[96:128]

    def r1(a):
        return jnp.reshape(a, (1, -1))

    row2 = jnp.reshape(row, (E // CH, CH))
    col2 = jnp.reshape(col, (E // CH, CH))
    zeros640 = jnp.zeros((640, L), F32)
    ones_e = jnp.ones((E, L), F32)
    cntp = _sc_scatter(ones_e, col2, zeros640)
    cnt2 = cntp[:, :N, :]

    enc_w = [
        p["enc_node"]["W1"], r1(p["enc_node"]["b1"]),
        p["enc_node"]["W2"], r1(p["enc_node"]["b2"]),
        r1(p["enc_node"]["ln_scale"]), r1(p["enc_node"]["ln_bias"]),
        p["enc_glob"]["W1"], r1(p["enc_glob"]["b1"]),
        p["enc_glob"]["W2"], r1(p["enc_glob"]["b2"]),
        A1, A2, B1, B2, N1, D1, D2, b1e, Tt,
    ]
    (v0, pa0, pb0, nv0, PA, PB, invb, g0, cb) = _enc_node_call(
        x, cnt2, global_attr, enc_w, interpret=interpret)

    ee_w = [
        _bd4(p["enc_edge"]["W1"]), _t4(p["enc_edge"]["b1"]),
        _bd4(p["enc_edge"]["W2"]), _t4(p["enc_edge"]["b2"]),
        _t4(p["enc_edge"]["ln_scale"]), _t4(p["enc_edge"]["ln_bias"]),
        Mavg, _bd4(C1),
    ]
    eap = jnp.reshape(edge_attr, (EP4, 64))
    e0p, ec0p = _enc_edge_call(eap, ee_w, interpret=interpret)

    es_w = [
        _bd4(C2), _bd4(p["core_edge"]["W2"]), _t4(p["core_edge"]["b2"]),
        _t4(p["core_edge"]["ln_scale"]), _t4(p["core_edge"]["ln_bias"]),
        _bd4(p["dec_edge"]["W1"]), _t4(p["dec_edge"]["b1"]),
        _bd4(p["dec_edge"]["W2"]), _t4(p["dec_edge"]["b2"]),
        _t4(p["dec_edge"]["ln_scale"]), _t4(p["dec_edge"]["ln_bias"]),
        Mavg,
    ]
    ns_w = [
        _bd4(N2), _bd4(N3), _bd4(p["core_node"]["W2"]), _t4(p["core_node"]["b2"]),
        _t4(p["core_node"]["ln_scale"]), _t4(p["core_node"]["ln_bias"]),
        _bd4(p["dec_node"]["W1"]), _t4(p["dec_node"]["b1"]),
        _bd4(p["dec_node"]["W2"]), _t4(p["dec_node"]["b2"]),
        _t4(p["dec_node"]["ln_scale"]), _t4(p["dec_node"]["ln_bias"]),
        Mavg, Pfold, Tt,
        Ng1, Ng2, b1n, G1, G2, G3, G4, r1(p["core_glob"]["b1"]),
        p["core_glob"]["W2"], r1(p["core_glob"]["b2"]),
        p["dec_glob"]["W1"], r1(p["dec_glob"]["b1"]),
        p["dec_glob"]["W2"], r1(p["dec_glob"]["b2"]),
        D1, D2, b1e, _bd4(A2), _bd4(B2),
    ]

    nv0p = jnp.reshape(nv0, (NP4, 128))
    pa0p = jnp.reshape(pa0, (NP4, 128))
    pb0p = jnp.reshape(pb0, (NP4, 128))
    invp = jnp.reshape(invb, (NP4, 128))
    vp = jnp.reshape(v0, (NP4, 128))
    ep = e0p
    g = g0
    for _ in range(3):
        sa, sb = _sc_gather(row2, col2, jnp.reshape(PA, (N, L)),
                            jnp.reshape(PB, (N, L)))
        sap = jnp.reshape(sa, (EP4, 128))
        sbp = jnp.reshape(sb, (EP4, 128))
        ecp, enp, esum = _edge_step_call(ep, ec0p, sap, sbp, cb, es_w,
                                         interpret=interpret)
        aggp = _sc_scatter(jnp.reshape(ecp, (E, L)), col2, zeros640)
        agg2 = jnp.reshape(aggp[:, :N, :], (2, NP4, 128))
        vp, PA, PB, g, cb = _node_step_call(
            vp, nv0p, pa0p, pb0p, agg2, invp, esum, g0, g, ns_w,
            interpret=interpret)
        ep = enp

    on_w = [
        p["out_node"]["W1"], r1(p["out_node"]["b1"]),
        p["out_node"]["W2"], r1(p["out_node"]["b2"]),
        p["out_glob"]["W1"], r1(p["out_glob"]["b1"]),
        p["out_glob"]["W2"], r1(p["out_glob"]["b2"]),
    ]
    out_v, out_g = _out_node_call(jnp.reshape(vp, (N, L)), g, on_w,
                                  interpret=interpret)
    oe_w = [
        _bd4(p["out_edge"]["W1"]), _t4(p["out_edge"]["b1"]),
        _bd4(p["out_edge"]["W2"]), _t4(p["out_edge"]["b2"]),
    ]
    (oep,) = _out_edge_call(ep, oe_w, interpret=interpret)
    out_e = jnp.reshape(oep, (E, 16))
    return (out_v, out_e, out_g)


def kernel(x, edge_index, edge_attr, global_attr, params):
    return _forward(x, edge_index, edge_attr, global_attr, params)


# speedup vs baseline: 5.6583x; 1.4157x over previous
import functools

import jax
import jax.numpy as jnp
from jax import lax
from jax.experimental import pallas as pl
from jax.experimental.pallas import tpu as pltpu
from jax.experimental.pallas import tpu_sc as plsc

F32 = jnp.float32
N = 10000
E = 160000
L = 32
EP4 = E // 4
NP4 = N // 4
BE = 2000
NPAD = 10240
NW = 32
CH = 128
CPW = 39


def _relu(x):
    return jnp.maximum(x, 0.0)


def _dot(a, b):
    return jnp.dot(a, b, preferred_element_type=F32)


def _dot_hi(a, b):
    return jnp.dot(a, b, preferred_element_type=F32,
                   precision=lax.Precision.HIGHEST)


def _dot_stat(a, b):
    ab = a.astype(jnp.bfloat16).astype(F32)
    return _dot(ab, b) + _dot(a - ab, b)


def _ln_packed(h, mavg, scale, bias):
    mu = _dot_stat(h, mavg)
    hc = h - mu
    var = _dot_stat(hc * hc, mavg)
    return hc * lax.rsqrt(var + 1e-5) * scale + bias


def _fa(shape):
    n = len(shape)
    return pl.BlockSpec(shape, lambda i: (0,) * n)


GG = 10


def _sc_gather(row2d, col2d, pa, pb):
    mesh = plsc.VectorSubcoreMesh(core_axis_name="c", subcore_axis_name="s")

    @functools.partial(
        pl.kernel,
        mesh=mesh,
        out_type=(jax.ShapeDtypeStruct((E, L), F32),
                  jax.ShapeDtypeStruct((E, L), F32)),
        compiler_params=pltpu.CompilerParams(use_tc_tiling_on_sc=False),
        scratch_types=[
            pltpu.VMEM((CPW + 1, CH), jnp.int32),
            pltpu.VMEM((GG * CH, L), F32),
            pltpu.VMEM((GG * CH, L), F32),
            pltpu.SemaphoreType.DMA,
            pltpu.SemaphoreType.DMA,
        ],
    )
    def k(row_h, col_h, pa_h, pb_h, sa_h, sb_h, idx, b0, b1, gsem, wsem):
        wid = lax.axis_index("s") * 2 + lax.axis_index("c")
        bufs = (b0, b1)

        def phase(src_h, tab_h, out_h, pending):
            pltpu.sync_copy(src_h.at[pl.ds(wid * CPW, CPW)],
                            idx.at[pl.ds(0, CPW)])

            @pl.when(wid < 2)
            def _():
                pltpu.sync_copy(src_h.at[pl.ds(NW * CPW + wid, 1)],
                                idx.at[pl.ds(CPW, 1)])
            for g in range((CPW + GG - 1) // GG):
                nch = min(GG, CPW - g * GG)
                buf = bufs[g % 2]
                copies = []
                for j in range(nch):
                    ch = g * GG + j
                    copies.append(pltpu.async_copy(
                        tab_h.at[idx.at[ch]],
                        buf.at[pl.ds(j * CH, CH)], gsem))
                if pending[0] is not None:
                    pending[0].wait()
                    pending[0] = None
                for cpy in copies:
                    cpy.wait()
                wb = pltpu.async_copy(
                    buf.at[pl.ds(0, nch * CH)],
                    out_h.at[pl.ds(wid * (CPW * CH) + g * GG * CH, nch * CH)],
                    wsem)
                pending[0] = wb

            @pl.when(wid < 2)
            def _():
                cpy = pltpu.async_copy(tab_h.at[idx.at[CPW]],
                                       bufs[0].at[pl.ds(0, CH)], gsem)
                cpy.wait()
                pltpu.sync_copy(bufs[0].at[pl.ds(0, CH)],
                                out_h.at[pl.ds((NW * CPW + wid) * CH, CH)])
            if pending[0] is not None:
                pending[0].wait()
                pending[0] = None

        pending = [None]
        phase(row_h, pa_h, sa_h, pending)
        phase(col_h, pb_h, sb_h, pending)

    return k(row2d, col2d, pa, pb)


SG = 20


def _sc_scatter(vals, col2d, zeros):
    mesh = plsc.VectorSubcoreMesh(core_axis_name="c", subcore_axis_name="s")

    @functools.partial(
        pl.kernel,
        mesh=mesh,
        out_type=jax.ShapeDtypeStruct((2, NPAD, L), F32),
        compiler_params=pltpu.CompilerParams(use_tc_tiling_on_sc=False),
        scratch_types=[
            pltpu.VMEM((CPW + 1, CH), jnp.int32),
            pltpu.VMEM((SG * CH, L), F32),
            pltpu.VMEM_SHARED((NPAD, L), F32),
            pltpu.SemaphoreType.DMA,
        ],
    )
    def k(vals_h, col_h, z_h, out_h, idx, buf, acc, ssem):
        c = lax.axis_index("c")
        s = lax.axis_index("s")
        base_ch = c * 625 + s * CPW
        pltpu.sync_copy(z_h, acc.at[pl.ds(s * 640, 640)])
        pltpu.sync_copy(col_h.at[pl.ds(base_ch, CPW)], idx.at[pl.ds(0, CPW)])

        @pl.when(s == 15)
        def _():
            pltpu.sync_copy(col_h.at[pl.ds(c * 625 + 624, 1)],
                            idx.at[pl.ds(CPW, 1)])
        plsc.subcore_barrier()
        for g in range((CPW + SG - 1) // SG):
            nch = min(SG, CPW - g * SG)
            pltpu.sync_copy(
                vals_h.at[pl.ds((base_ch + g * SG) * CH, nch * CH)],
                buf.at[pl.ds(0, nch * CH)])
            copies = []
            for j in range(nch):
                copies.append(pltpu.async_copy(
                    buf.at[pl.ds(j * CH, CH)],
                    acc.at[idx.at[g * SG + j]], ssem, add=True))
            for cpy in copies:
                cpy.wait()

        @pl.when(s == 15)
        def _():
            pltpu.sync_copy(vals_h.at[pl.ds((c * 625 + 624) * CH, CH)],
                            buf.at[pl.ds(0, CH)])
            pltpu.sync_copy(buf.at[pl.ds(0, CH)], acc.at[idx.at[CPW]], add=True)
        plsc.subcore_barrier()
        pltpu.sync_copy(acc.at[pl.ds(s * 640, 640)], out_h.at[c, pl.ds(s * 640, 640)])

    return k(vals, col2d, zeros)


def _enc_node_call(x, cnt2, ga, w, interpret=False):
    def body(x_r, cnt_r, ga_r,
             enW1, enb1, enW2, enb2, enls, enlb,
             egW1, egb1, egW2, egb2,
             A1, A2, B1, B2, N1, D1, D2, b1e, Tt,
             v0_r, pa0_r, pb0_r, nv0_r, PA1_r, PB1_r, inv_r, g0_r, cb1_r):
        h = _relu(_dot(x_r[...], enW1[...]) + enb1[...])
        h = _dot(h, enW2[...]) + enb2[...]
        h = _relu(h)
        mu = jnp.mean(h, axis=1, keepdims=True)
        hc = h - mu
        var = jnp.mean(hc * hc, axis=1, keepdims=True)
        v0 = hc * lax.rsqrt(var + 1e-5) * enls[...] + enlb[...]
        v0_r[...] = v0
        g = _relu(_dot(ga_r[...], egW1[...]) + egb1[...])
        g0 = _relu(_dot(g, egW2[...]) + egb2[...])
        g0_r[...] = g0
        pa0 = _dot(v0, A1[...])
        pb0 = _dot(v0, B1[...])
        pa0_r[...] = pa0
        pb0_r[...] = pb0
        nv0_r[...] = _dot(v0, N1[...])
        PA1_r[...] = pa0 + _dot(v0, A2[...])
        PB1_r[...] = pb0 + _dot(v0, B2[...])
        cnt = cnt_r[0, :, 0:1] + cnt_r[1, :, 0:1]
        inv_r[...] = jnp.broadcast_to(1.0 / jnp.maximum(cnt, 1.0), inv_r.shape)
        cb1_r[...] = _dot_hi(
            _dot(g0, D1[...]) + _dot(g0, D2[...]) + b1e[...], Tt[...])

    BN = 2000
    grid = (N // BN,)
    in_specs = ([pl.BlockSpec((BN, 128), lambda i: (i, 0)),
                 pl.BlockSpec((2, BN, L), lambda i: (0, i, 0)),
                 _fa(ga.shape)] + [_fa(a.shape) for a in w])
    out_specs = [pl.BlockSpec((BN, L), lambda i: (i, 0))] * 7 + [
        pl.BlockSpec((1, L), lambda i: (0, 0)),
        pl.BlockSpec((1, 128), lambda i: (0, 0)),
    ]
    outs = [jax.ShapeDtypeStruct((N, L), F32)] * 7 + [
        jax.ShapeDtypeStruct((1, L), F32),
        jax.ShapeDtypeStruct((1, 128), F32),
    ]
    return pl.pallas_call(body, grid=grid, in_specs=in_specs,
                          out_specs=out_specs, out_shape=outs,
                          interpret=interpret)(x, cnt2, ga, *w)


def _enc_edge_call(eap, w, interpret=False):
    def body(ea_r, W1p, b1t, W2p, b2t, lst, lbt, Mavg, C1p, e0_r, ec0_r):
        h = _relu(_dot(ea_r[...], W1p[...]) + b1t[...])
        h = _dot(h, W2p[...]) + b2t[...]
        h = _relu(h)
        e0 = _ln_packed(h, Mavg[...], lst[...], lbt[...])
        e0_r[...] = e0
        ec0_r[...] = _dot(e0, C1p[...])

    grid = (EP4 // BE,)
    in_specs = [pl.BlockSpec((BE, 64), lambda i: (i, 0))] + [_fa(a.shape) for a in w]
    out_specs = [pl.BlockSpec((BE, 128), lambda i: (i, 0))] * 2
    outs = [jax.ShapeDtypeStruct((EP4, 128), F32)] * 2
    return pl.pallas_call(body, grid=grid, in_specs=in_specs,
                          out_specs=out_specs, out_shape=outs,
                          interpret=interpret)(eap, *w)


def _edge_step_call(ep, ec0p, sap, sbp, cb, w, interpret=False):
    def body(ep_r, ec0_r, sa_r, sb_r, cb_r,
             C2p, W2p, b2t, lst, lbt, dW1p, db1t, dW2p, db2t, dlst, dlbt, Mavg,
             ecp_r, enp_r, esum_r):
        t = _dot(ep_r[...], C2p[...]) + ec0_r[...] + sa_r[...] + sb_r[...] + cb_r[...]
        h1 = _relu(t)
        h2 = _dot(h1, W2p[...]) + b2t[...]
        h2 = _relu(h2)
        ec = _ln_packed(h2, Mavg[...], lst[...], lbt[...])
        ecp_r[...] = ec

        @pl.when(pl.program_id(0) == 0)
        def _():
            esum_r[...] = jnp.zeros_like(esum_r)

        esum_r[...] += jnp.sum(ec, axis=0, keepdims=True)
        d = _relu(_dot(ec, dW1p[...]) + db1t[...])
        d = _dot(d, dW2p[...]) + db2t[...]
        d = _relu(d)
        enp_r[...] = _ln_packed(d, Mavg[...], dlst[...], dlbt[...])

    grid = (EP4 // BE,)
    in_specs = ([pl.BlockSpec((BE, 128), lambda i: (i, 0))] * 4
                + [_fa(cb.shape)] + [_fa(a.shape) for a in w])
    out_specs = [pl.BlockSpec((BE, 128), lambda i: (i, 0))] * 2 + [
        pl.BlockSpec((1, 128), lambda i: (0, 0))]
    outs = [jax.ShapeDtypeStruct((EP4, 128), F32)] * 2 + [
        jax.ShapeDtypeStruct((1, 128), F32)]
    return pl.pallas_call(body, grid=grid, in_specs=in_specs,
                          out_specs=out_specs, out_shape=outs,
                          interpret=interpret)(ep, ec0p, sap, sbp, cb, *w)


def _node_step_call(vp, nv0p, pa0p, pb0p, agg2, invp, esum, g0, g, w,
                    interpret=False):
    def body(vp_r, nv0_r, pa0_r, pb0_r, agg2_r, inv_r, esum_r, g0_r, g_r,
             N2p, N3p, nW2p, nb2t, nlst, nlbt,
             dnW1p, dnb1t, dnW2p, dnb2t, dnlst, dnlbt,
             Mavg, Pfold, Tt,
             Ng1, Ng2, b1n, G1, G2, G3, G4, b1g, gW2, gb2,
             dgW1, dgb1, dgW2, dgb2, D1, D2, b1e, A2p, B2p,
             vn_r, PA_r, PB_r, gn_r, cbn_r):
        g0 = g0_r[...]
        g = g_r[...]
        agg = (agg2_r[0] + agg2_r[1]) * inv_r[...]
        gb = _dot_hi(_dot(g0, Ng1[...]) + _dot(g, Ng2[...]) + b1n[...], Tt[...])
        n1 = nv0_r[...] + _dot(vp_r[...], N2p[...]) + _dot(agg, N3p[...]) + gb
        h = _relu(n1)
        h2 = _dot(h, nW2p[...]) + nb2t[...]
        h2 = _relu(h2)
        vc = _ln_packed(h2, Mavg[...], nlst[...], nlbt[...])
        vsum = jnp.sum(vc, axis=0, keepdims=True)
        meanv = _dot_hi(vsum, Pfold[...]) / N
        meane = _dot_hi(esum_r[...], Pfold[...]) / E
        g1 = (_dot(g0, G1[...]) + _dot(g, G2[...]) + _dot(meanv, G3[...])
              + _dot(meane, G4[...]) + b1g[...])
        gc = _relu(_dot(_relu(g1), gW2[...]) + gb2[...])
        gn = _relu(_dot(_relu(_dot(gc, dgW1[...]) + dgb1[...]), dgW2[...])
                   + dgb2[...])
        gn_r[...] = gn
        d = _relu(_dot(vc, dnW1p[...]) + dnb1t[...])
        d = _dot(d, dnW2p[...]) + dnb2t[...]
        d = _relu(d)
        vn = _ln_packed(d, Mavg[...], dnlst[...], dnlbt[...])
        vn_r[...] = vn
        PA_r[...] = pa0_r[...] + _dot(vn, A2p[...])
        PB_r[...] = pb0_r[...] + _dot(vn, B2p[...])
        cbn_r[...] = _dot_hi(_dot(g0, D1[...]) + _dot(gn, D2[...]) + b1e[...],
                             Tt[...])

    outs = [jax.ShapeDtypeStruct((NP4, 128), F32)] * 3 + [
        jax.ShapeDtypeStruct((1, L), F32),
        jax.ShapeDtypeStruct((1, 128), F32),
    ]
    return pl.pallas_call(body, out_shape=outs, interpret=interpret)(
        vp, nv0p, pa0p, pb0p, agg2, invp, esum, g0, g, *w)


def _out_node_call(v, g, w, interpret=False):
    def body(v_r, g_r, W1, b1, W2, b2, gW1, gb1, gW2, gb2, ov_r, og_r):
        h = _relu(_dot(v_r[...], W1[...]) + b1[...])
        ov_r[...] = _dot(h, W2[...]) + b2[...]
        hg = _relu(_dot(g_r[...], gW1[...]) + gb1[...])
        og_r[...] = _dot(hg, gW2[...]) + gb2[...]

    outs = [jax.ShapeDtypeStruct((N, 128), F32),
            jax.ShapeDtypeStruct((1, 16), F32)]
    return pl.pallas_call(body, out_shape=outs, interpret=interpret)(v, g, *w)


def _out_edge_call(ep, w, interpret=False):
    def body(ep_r, W1p, b1t, W2p, b2t, oe_r):
        h = _relu(_dot(ep_r[...], W1p[...]) + b1t[...])
        oe_r[...] = _dot(h, W2p[...]) + b2t[...]

    grid = (EP4 // BE,)
    in_specs = [pl.BlockSpec((BE, 128), lambda i: (i, 0))] + [
        _fa(a.shape) for a in w]
    out_specs = [pl.BlockSpec((BE, 64), lambda i: (i, 0))]
    outs = [jax.ShapeDtypeStruct((EP4, 64), F32)]
    return pl.pallas_call(body, grid=grid, in_specs=in_specs,
                          out_specs=out_specs, out_shape=outs,
                          interpret=interpret)(ep, *w)


def _bd4(wmat):
    return jax.scipy.linalg.block_diag(wmat, wmat, wmat, wmat)


def _t4(b):
    return jnp.tile(jnp.reshape(b, (1, -1)), (1, 4))


def _forward(x, edge_index, edge_attr, global_attr, params, interpret=False):
    p = params
    row = edge_index[0]
    col = edge_index[1]
    eye = jnp.eye(L, dtype=F32)
    Tt = jnp.concatenate([eye] * 4, axis=1)
    Pfold = jnp.concatenate([eye] * 4, axis=0)
    Mavg = _bd4(jnp.full((L, L), 1.0 / L, F32))

    We = p["core_edge"]["W1"]
    A1, A2 = We[0:32], We[32:64]
    B1, B2 = We[64:96], We[96:128]
    C1, C2 = We[128:160], We[160:192]
    D1, D2 = We[192:224], We[224:256]
    b1e = jnp.reshape(p["core_edge"]["b1"], (1, L))
    Wn = p["core_node"]["W1"]
    N1, N2, N3, Ng1, Ng2 = Wn[0:32], Wn[32:64], Wn[64:96], Wn[96:128], Wn[128:160]
    b1n = jnp.reshape(p["core_node"]["b1"], (1, L))
    Wg = p["core_glob"]["W1"]
    G1, G2, G3, G4 = Wg[0:32], Wg[32:64], Wg[64:96], Wg[96:128]

    def r1(a):
        return jnp.reshape(a, (1, -1))

    row2 = jnp.reshape(row, (E // CH, CH))
    col2 = jnp.reshape(col, (E // CH, CH))
    zeros640 = jnp.zeros((640, L), F32)
    ones_e = jnp.ones((E, L), F32)
    cntp = _sc_scatter(ones_e, col2, zeros640)
    cnt2 = cntp[:, :N, :]

    enc_w = [
        p["enc_node"]["W1"], r1(p["enc_node"]["b1"]),
        p["enc_node"]["W2"], r1(p["enc_node"]["b2"]),
        r1(p["enc_node"]["ln_scale"]), r1(p["enc_node"]["ln_bias"]),
        p["enc_glob"]["W1"], r1(p["enc_glob"]["b1"]),
        p["enc_glob"]["W2"], r1(p["enc_glob"]["b2"]),
        A1, A2, B1, B2, N1, D1, D2, b1e, Tt,
    ]
    (v0, pa0, pb0, nv0, PA, PB, invb, g0, cb) = _enc_node_call(
        x, cnt2, global_attr, enc_w, interpret=interpret)

    ee_w = [
        _bd4(p["enc_edge"]["W1"]), _t4(p["enc_edge"]["b1"]),
        _bd4(p["enc_edge"]["W2"]), _t4(p["enc_edge"]["b2"]),
        _t4(p["enc_edge"]["ln_scale"]), _t4(p["enc_edge"]["ln_bias"]),
        Mavg, _bd4(C1),
    ]
    eap = jnp.reshape(edge_attr, (EP4, 64))
    e0p, ec0p = _enc_edge_call(eap, ee_w, interpret=interpret)

    es_w = [
        _bd4(C2), _bd4(p["core_edge"]["W2"]), _t4(p["core_edge"]["b2"]),
        _t4(p["core_edge"]["ln_scale"]), _t4(p["core_edge"]["ln_bias"]),
        _bd4(p["dec_edge"]["W1"]), _t4(p["dec_edge"]["b1"]),
        _bd4(p["dec_edge"]["W2"]), _t4(p["dec_edge"]["b2"]),
        _t4(p["dec_edge"]["ln_scale"]), _t4(p["dec_edge"]["ln_bias"]),
        Mavg,
    ]
    ns_w = [
        _bd4(N2), _bd4(N3), _bd4(p["core_node"]["W2"]), _t4(p["core_node"]["b2"]),
        _t4(p["core_node"]["ln_scale"]), _t4(p["core_node"]["ln_bias"]),
        _bd4(p["dec_node"]["W1"]), _t4(p["dec_node"]["b1"]),
        _bd4(p["dec_node"]["W2"]), _t4(p["dec_node"]["b2"]),
        _t4(p["dec_node"]["ln_scale"]), _t4(p["dec_node"]["ln_bias"]),
        Mavg, Pfold, Tt,
        Ng1, Ng2, b1n, G1, G2, G3, G4, r1(p["core_glob"]["b1"]),
        p["core_glob"]["W2"], r1(p["core_glob"]["b2"]),
        p["dec_glob"]["W1"], r1(p["dec_glob"]["b1"]),
        p["dec_glob"]["W2"], r1(p["dec_glob"]["b2"]),
        D1, D2, b1e, _bd4(A2), _bd4(B2),
    ]

    nv0p = jnp.reshape(nv0, (NP4, 128))
    pa0p = jnp.reshape(pa0, (NP4, 128))
    pb0p = jnp.reshape(pb0, (NP4, 128))
    invp = jnp.reshape(invb, (NP4, 128))
    vp = jnp.reshape(v0, (NP4, 128))
    ep = e0p
    g = g0
    for _ in range(3):
        sa, sb = _sc_gather(row2, col2, jnp.reshape(PA, (N, L)),
                            jnp.reshape(PB, (N, L)))
        sap = jnp.reshape(sa, (EP4, 128))
        sbp = jnp.reshape(sb, (EP4, 128))
        ecp, enp, esum = _edge_step_call(ep, ec0p, sap, sbp, cb, es_w,
                                         interpret=interpret)
        aggp = _sc_scatter(jnp.reshape(ecp, (E, L)), col2, zeros640)
        agg2 = jnp.reshape(aggp[:, :N, :], (2, NP4, 128))
        vp, PA, PB, g, cb = _node_step_call(
            vp, nv0p, pa0p, pb0p, agg2, invp, esum, g0, g, ns_w,
            interpret=interpret)
        ep = enp

    on_w = [
        p["out_node"]["W1"], r1(p["out_node"]["b1"]),
        p["out_node"]["W2"], r1(p["out_node"]["b2"]),
        p["out_glob"]["W1"], r1(p["out_glob"]["b1"]),
        p["out_glob"]["W2"], r1(p["out_glob"]["b2"]),
    ]
    out_v, out_g = _out_node_call(jnp.reshape(vp, (N, L)), g, on_w,
                                  interpret=interpret)
    oe_w = [
        _bd4(p["out_edge"]["W1"]), _t4(p["out_edge"]["b1"]),
        _bd4(p["out_edge"]["W2"]), _t4(p["out_edge"]["b2"]),
    ]
    (oep,) = _out_edge_call(ep, oe_w, interpret=interpret)
    out_e = jnp.reshape(oep, (E, 16))
    return (out_v, out_e, out_g)


def kernel(x, edge_index, edge_attr, global_attr, params):
    return _forward(x, edge_index, edge_attr, global_attr, params)
